# Initial kernel scaffold; baseline (speedup 1.0000x reference)
import jax
import jax.numpy as jnp
from jax.experimental import pallas as pl


def kernel(x, fc_w, fc_b, centroids):
    raise NotImplementedError("write your pallas kernel here")


# speedup vs baseline: 2.9001x; 2.9001x over previous
import jax
import jax.numpy as jnp
from jax.experimental import pallas as pl
from jax.experimental.pallas import tpu as pltpu

_N, _C, _S, _K = 16, 128, 16384, 64
_CS = 2048
_EPS = 1e-12
_EPS2 = _EPS * _EPS


def _netvlad_body(x_ref, w_ref, b_ref, cent_ref, out_ref):
    w = w_ref[...]
    b = b_ref[...]

    vlad = jnp.zeros((_K, _C), jnp.float32)
    asum = jnp.zeros((_K, 1), jnp.float32)

    for ci in range(_S // _CS):
        xb = x_ref[0, :, ci * _CS:(ci + 1) * _CS]
        ss = jnp.sum(xb * xb, axis=0, keepdims=True)
        inv = jax.lax.rsqrt(jnp.maximum(ss, _EPS2))
        raw = jnp.dot(w, xb, preferred_element_type=jnp.float32)
        logits = raw * inv + b
        m = jnp.max(logits, axis=0, keepdims=True)
        e = jnp.exp(logits - m)
        denom = jnp.sum(e, axis=0, keepdims=True)
        p = e * (1.0 / denom)
        asum = asum + jnp.sum(p, axis=1, keepdims=True)
        q = p * inv
        vlad = vlad + jax.lax.dot_general(
            q, xb, (((1,), (1,)), ((), ())),
            preferred_element_type=jnp.float32)

    v = vlad - asum * cent_ref[...]
    ssr = jnp.sum(v * v, axis=1, keepdims=True)
    v = v * jax.lax.rsqrt(jnp.maximum(ssr, _EPS2))
    tot = jnp.sum(jnp.sum(v * v, axis=1, keepdims=True), axis=0, keepdims=True)
    v = v * jax.lax.rsqrt(jnp.maximum(tot, _EPS2))
    out_ref[0] = v


def kernel(x, fc_w, fc_b, centroids):
    out = pl.pallas_call(
        _netvlad_body,
        out_shape=jax.ShapeDtypeStruct((_N, _K, _C), jnp.float32),
        grid=(_N,),
        in_specs=[
            pl.BlockSpec((1, _C, _S), lambda n: (n, 0, 0)),
            pl.BlockSpec((_K, _C), lambda n: (0, 0)),
            pl.BlockSpec((_K, 1), lambda n: (0, 0)),
            pl.BlockSpec((_K, _C), lambda n: (0, 0)),
        ],
        out_specs=pl.BlockSpec((1, _K, _C), lambda n: (n, 0, 0)),
        compiler_params=pltpu.CompilerParams(
            dimension_semantics=("parallel",),
            vmem_limit_bytes=48 * 1024 * 1024,
        ),
        name="netvlad_fused",
    )(x, fc_w, fc_b.reshape(_K, 1), centroids)
    return out.reshape(_N, _K * _C)


# speedup vs baseline: 3.1677x; 1.0923x over previous
import jax
import jax.numpy as jnp
from jax.experimental import pallas as pl
from jax.experimental.pallas import tpu as pltpu

_N, _C, _S, _K = 16, 128, 16384, 64
_CA = _C + 8
_CS = 2048
_EPS = 1e-12
_EPS2 = _EPS * _EPS


def _netvlad_body(x_ref, wa_ref, cent_ref, out_ref):
    wa = wa_ref[...]

    ones_blk = jnp.where(
        jax.lax.broadcasted_iota(jnp.int32, (8, _CS), 0) == 0, 1.0, 0.0
    ).astype(jnp.float32)

    acc = jnp.zeros((_K, _CA), jnp.float32)

    for ci in range(_S // _CS):
        xb = x_ref[0, :, ci * _CS:(ci + 1) * _CS]
        ss = jnp.sum(xb * xb, axis=0, keepdims=True)
        inv = jax.lax.rsqrt(jnp.maximum(ss, _EPS2))
        xn = xb * inv
        xn_aug = jnp.concatenate([xn, ones_blk], axis=0)
        logits = jnp.dot(wa, xn_aug, preferred_element_type=jnp.float32)
        m = jnp.max(logits, axis=0, keepdims=True)
        e = jnp.exp(logits - m)
        denom = jnp.sum(e, axis=0, keepdims=True)
        p = e * (1.0 / denom)
        acc = acc + jax.lax.dot_general(
            p, xn_aug, (((1,), (1,)), ((), ())),
            preferred_element_type=jnp.float32)

    vlad = acc[:, :_C]
    asum = acc[:, _C:_C + 1]
    v = vlad - asum * cent_ref[...]
    ssr = jnp.sum(v * v, axis=1, keepdims=True)
    v = v * jax.lax.rsqrt(jnp.maximum(ssr, _EPS2))
    tot = jnp.sum(jnp.sum(v * v, axis=1, keepdims=True), axis=0, keepdims=True)
    v = v * jax.lax.rsqrt(jnp.maximum(tot, _EPS2))
    out_ref[0] = v


def kernel(x, fc_w, fc_b, centroids):
    w_aug = jnp.concatenate(
        [fc_w, fc_b[:, None], jnp.zeros((_K, _CA - _C - 1), jnp.float32)],
        axis=1)
    out = pl.pallas_call(
        _netvlad_body,
        out_shape=jax.ShapeDtypeStruct((_N, _K, _C), jnp.float32),
        grid=(_N,),
        in_specs=[
            pl.BlockSpec((1, _C, _S), lambda n: (n, 0, 0)),
            pl.BlockSpec((_K, _CA), lambda n: (0, 0)),
            pl.BlockSpec((_K, _C), lambda n: (0, 0)),
        ],
        out_specs=pl.BlockSpec((1, _K, _C), lambda n: (n, 0, 0)),
        compiler_params=pltpu.CompilerParams(
            dimension_semantics=("parallel",),
            vmem_limit_bytes=48 * 1024 * 1024,
        ),
        name="netvlad_fused",
    )(x, w_aug, centroids)
    return out.reshape(_N, _K * _C)


# speedup vs baseline: 3.3759x; 1.0657x over previous
import jax
import jax.numpy as jnp
from jax.experimental import pallas as pl
from jax.experimental.pallas import tpu as pltpu

_N, _C, _S, _K = 16, 128, 16384, 64
_CA = _C + 8
_CS = 2048
_EPS = 1e-12
_EPS2 = _EPS * _EPS


def _netvlad_body(x_ref, wa_ref, cent_ref, out_ref):
    wa = wa_ref[...]
    row0 = jax.lax.broadcasted_iota(jnp.int32, (8, _CS), 0) == 0

    acc = jnp.zeros((_K, _CA), jnp.float32)

    for ci in range(_S // _CS):
        xb = x_ref[0, :, ci * _CS:(ci + 1) * _CS]
        ss = jnp.sum(xb * xb, axis=0, keepdims=True)
        ssc = jnp.maximum(ss, _EPS2)
        inv = jax.lax.rsqrt(ssc)
        nrm = jnp.sqrt(ssc)
        nrm_blk = jnp.where(row0, nrm, 0.0)
        xb_aug = jnp.concatenate([xb, nrm_blk], axis=0)
        raw = jnp.dot(wa, xb_aug, preferred_element_type=jnp.float32)
        logits = raw * inv
        m = jnp.max(logits, axis=0, keepdims=True)
        e = jnp.exp(logits - m)
        denom = jnp.sum(e, axis=0, keepdims=True)
        q = e * ((1.0 / denom) * inv)
        acc = acc + jax.lax.dot_general(
            q, xb_aug, (((1,), (1,)), ((), ())),
            preferred_element_type=jnp.float32)

    vlad = acc[:, :_C]
    asum = acc[:, _C:_C + 1]
    v = vlad - asum * cent_ref[...]
    ssr = jnp.sum(v * v, axis=1, keepdims=True)
    v = v * jax.lax.rsqrt(jnp.maximum(ssr, _EPS2))
    tot = jnp.sum(jnp.sum(v * v, axis=1, keepdims=True), axis=0, keepdims=True)
    v = v * jax.lax.rsqrt(jnp.maximum(tot, _EPS2))
    out_ref[0] = v


def kernel(x, fc_w, fc_b, centroids):
    w_aug = jnp.concatenate(
        [fc_w, fc_b[:, None], jnp.zeros((_K, _CA - _C - 1), jnp.float32)],
        axis=1)
    out = pl.pallas_call(
        _netvlad_body,
        out_shape=jax.ShapeDtypeStruct((_N, _K, _C), jnp.float32),
        grid=(_N,),
        in_specs=[
            pl.BlockSpec((1, _C, _S), lambda n: (n, 0, 0)),
            pl.BlockSpec((_K, _CA), lambda n: (0, 0)),
            pl.BlockSpec((_K, _C), lambda n: (0, 0)),
        ],
        out_specs=pl.BlockSpec((1, _K, _C), lambda n: (n, 0, 0)),
        compiler_params=pltpu.CompilerParams(
            dimension_semantics=("parallel",),
            vmem_limit_bytes=48 * 1024 * 1024,
        ),
        name="netvlad_fused",
    )(x, w_aug, centroids)
    return out.reshape(_N, _K * _C)


# speedup vs baseline: 3.6645x; 1.0855x over previous
import jax
import jax.numpy as jnp
from jax.experimental import pallas as pl
from jax.experimental.pallas import tpu as pltpu

_N, _C, _S, _K = 16, 128, 16384, 64
_CA = _C + 8
_CS = 2048
_NB = 2
_EPS = 1e-12
_EPS2 = _EPS * _EPS
_LOG2E = 1.4426950408889634


def _netvlad_body(x_ref, wa_ref, cent_ref, out_ref):
    wa = wa_ref[...]
    cent = cent_ref[...]
    row0 = jax.lax.broadcasted_iota(jnp.int32, (8, _CS), 0) == 0

    for nb in range(_NB):
        acc = jnp.zeros((_K, _CA), jnp.float32)
        for ci in range(_S // _CS):
            xb = x_ref[nb, :, ci * _CS:(ci + 1) * _CS]
            ss = jnp.sum(xb * xb, axis=0, keepdims=True)
            ssc = jnp.maximum(ss, _EPS2)
            inv = jax.lax.rsqrt(ssc)
            nrm = jnp.sqrt(ssc)
            nrm_blk = jnp.where(row0, nrm, 0.0)
            xb_aug = jnp.concatenate([xb, nrm_blk], axis=0)
            raw = jnp.dot(wa, xb_aug, preferred_element_type=jnp.float32)
            m = jnp.max(raw, axis=0, keepdims=True)
            e = jnp.exp2((raw - m) * (inv * _LOG2E))
            denom = jnp.sum(e, axis=0, keepdims=True)
            q = e * ((1.0 / denom) * inv)
            acc = acc + jax.lax.dot_general(
                q, xb_aug, (((1,), (1,)), ((), ())),
                preferred_element_type=jnp.float32)

        vlad = acc[:, :_C]
        asum = acc[:, _C:_C + 1]
        v = vlad - asum * cent
        ssr = jnp.sum(v * v, axis=1, keepdims=True)
        v = v * jax.lax.rsqrt(jnp.maximum(ssr, _EPS2))
        tot = jnp.sum(jnp.sum(v * v, axis=1, keepdims=True),
                      axis=0, keepdims=True)
        v = v * jax.lax.rsqrt(jnp.maximum(tot, _EPS2))
        out_ref[nb] = v


def kernel(x, fc_w, fc_b, centroids):
    w_aug = jnp.concatenate(
        [fc_w, fc_b[:, None], jnp.zeros((_K, _CA - _C - 1), jnp.float32)],
        axis=1)
    out = pl.pallas_call(
        _netvlad_body,
        out_shape=jax.ShapeDtypeStruct((_N, _K, _C), jnp.float32),
        grid=(_N // _NB,),
        in_specs=[
            pl.BlockSpec((_NB, _C, _S), lambda n: (n, 0, 0)),
            pl.BlockSpec((_K, _CA), lambda n: (0, 0)),
            pl.BlockSpec((_K, _C), lambda n: (0, 0)),
        ],
        out_specs=pl.BlockSpec((_NB, _K, _C), lambda n: (n, 0, 0)),
        compiler_params=pltpu.CompilerParams(
            dimension_semantics=("parallel",),
            vmem_limit_bytes=52 * 1024 * 1024,
        ),
        name="netvlad_fused",
    )(x, w_aug, centroids)
    return out.reshape(_N, _K * _C)
